# baseline (speedup 1.0000x reference)
import jax
import jax.numpy as jnp
from jax import lax
from jax.experimental import pallas as pl
from jax.experimental.pallas import tpu as pltpu
from jax.experimental.pallas import tpu_sc as plsc

N = 10000
E = 320000
D = 128

NC = 2
NS = 16
NW = NC * NS
EPW = E // NW
NSG = 10
GE = 1024
EPWP = NSG * GE
NSGB = 5
GEB = 2000
CK = 80
SGB = GEB // CK
STRIPE = 624
ZR = 16


def _k1_body(x_ref, w_ref, asrc_ref, adst_ref, h_ref, s_ref, d_ref, m_ref):
    i = pl.program_id(0)
    h = jnp.dot(x_ref[...], w_ref[...], preferred_element_type=jnp.float32)
    h_ref[...] = h
    s = jnp.dot(h, asrc_ref[...], preferred_element_type=jnp.float32)
    d = jnp.dot(h, adst_ref[...], preferred_element_type=jnp.float32)
    s_ref[...] = s[:, None]
    d_ref[...] = d[:, None]
    bs = jnp.max(s)
    bd = jnp.max(d)

    @pl.when(i == 0)
    def _init():
        m_ref[0, 0] = bs
        m_ref[0, 1] = bd

    @pl.when(i > 0)
    def _acc():
        m_ref[0, 0] = jnp.maximum(m_ref[0, 0], bs)
        m_ref[0, 1] = jnp.maximum(m_ref[0, 1], bd)


_K1_BLK = 1000


def _k1(x, W, a_src, a_dst):
    return pl.pallas_call(
        _k1_body,
        grid=(N // _K1_BLK,),
        in_specs=[
            pl.BlockSpec((_K1_BLK, D), lambda i: (i, 0)),
            pl.BlockSpec((D, D), lambda i: (0, 0)),
            pl.BlockSpec((D,), lambda i: (0,)),
            pl.BlockSpec((D,), lambda i: (0,)),
        ],
        out_specs=[
            pl.BlockSpec((_K1_BLK, D), lambda i: (i, 0)),
            pl.BlockSpec((_K1_BLK, 1), lambda i: (i, 0)),
            pl.BlockSpec((_K1_BLK, 1), lambda i: (i, 0)),
            pl.BlockSpec((1, 16), lambda i: (0, 0), memory_space=pltpu.SMEM),
        ],
        out_shape=[
            jax.ShapeDtypeStruct((N, D), jnp.float32),
            jax.ShapeDtypeStruct((N, 1), jnp.float32),
            jax.ShapeDtypeStruct((N, 1), jnp.float32),
            jax.ShapeDtypeStruct((1, 16), jnp.float32),
        ],
    )(x, W, a_src, a_dst)


def _k2a_body(src_hbm, dst_hbm, s_hbm, d_hbm, m_hbm,
              p_hbm, den_hbm,
              src_v, dst_v, s_v, d_v, den_v, p_v, m_v):
    cid = lax.axis_index("c")
    sid = lax.axis_index("s")
    wid = sid * NC + cid

    pltpu.sync_copy(s_hbm, s_v)
    pltpu.sync_copy(d_hbm, d_v)
    pltpu.sync_copy(m_hbm, m_v)
    mvec = m_v[0, pl.ds(0, 16)]
    c_sh = jnp.maximum(mvec[0] + mvec[1], 0.0)

    zeros16 = jnp.zeros((16,), jnp.float32)

    def _zden(i, carry):
        den_v[0, pl.ds(i * 16, 16)] = zeros16
        return carry
    lax.fori_loop(0, N // 16, _zden, 0)

    lane = lax.iota(jnp.int32, 16)

    def _group(g, carry):
        pltpu.sync_copy(src_hbm.at[wid, g], src_v)
        pltpu.sync_copy(dst_hbm.at[wid, g], dst_v)

        def _row(r, carry2):
            for u in range(8):
                sl = pl.ds(u * 16, 16)
                si = src_v[r, sl]
                di = dst_v[r, sl]
                z = plsc.load_gather(s_v, [si]) + plsc.load_gather(d_v, [di])
                e = jnp.where(z > 0, z, z * jnp.float32(0.2))
                eidx = g * GE + r * 128 + u * 16 + lane
                p = jnp.where(eidx < EPW, jnp.exp(e - c_sh), 0.0)
                p_v[r, sl] = p
                plsc.addupdate_scatter(den_v.at[0], [di], p)
            return carry2
        lax.fori_loop(0, GE // 128, _row, 0)

        pltpu.sync_copy(p_v, p_hbm.at[wid, g])
        return carry

    lax.fori_loop(0, NSG, _group, 0)
    pltpu.sync_copy(den_v, den_hbm.at[wid])


def _k2a(src4, dst4, s, d, m):
    mesh = plsc.VectorSubcoreMesh(core_axis_name="c", subcore_axis_name="s")
    f = pl.kernel(
        _k2a_body,
        out_type=[
            jax.ShapeDtypeStruct((NW, NSG, GE // 128, 128), jnp.float32),
            jax.ShapeDtypeStruct((NW, 1, N), jnp.float32),
        ],
        mesh=mesh,
        scratch_types=[
            pltpu.VMEM((GE // 128, 128), jnp.int32),
            pltpu.VMEM((GE // 128, 128), jnp.int32),
            pltpu.VMEM((N,), jnp.float32),
            pltpu.VMEM((N,), jnp.float32),
            pltpu.VMEM((1, N), jnp.float32),
            pltpu.VMEM((GE // 128, 128), jnp.float32),
            pltpu.VMEM((1, 16), jnp.float32),
        ],
        compiler_params=pltpu.CompilerParams(needs_layout_passes=False),
    )
    return f(src4, dst4, s, d, m)


def _k2b_body(src_hbm, dst_hbm, p_hbm, h_hbm,
              parts_hbm,
              src_v, dst_v, p_v,
              rows0, rows1, acc,
              g0, g1, s0, s1):
    cid = lax.axis_index("c")
    sid = lax.axis_index("s")
    wid = sid * NC + cid
    base = pl.multiple_of(sid * STRIPE, 8)
    rows = [rows0, rows1]
    gsem = [g0, g1]
    ssem = [s0, s1]

    zeros16 = jnp.zeros((16,), jnp.float32)

    def _zbuf(i, carry):
        for t in range(D // 16):
            rows0[i, pl.ds(t * 16, 16)] = zeros16
        return carry
    lax.fori_loop(0, ZR, _zbuf, 0)
    zsrc = rows0.at[pl.ds(0, ZR)]

    def _zacc(r, carry):
        off = pl.multiple_of(base + r * ZR, 8)
        pltpu.sync_copy(zsrc, acc.at[pl.ds(off, ZR)])
        return carry
    lax.fori_loop(0, STRIPE // ZR, _zacc, 0)

    @pl.when(sid == NS - 1)
    def _ztail():
        pltpu.sync_copy(zsrc, acc.at[pl.ds(NS * STRIPE, N - NS * STRIPE)])

    plsc.subcore_barrier()

    def _scale(ci, rows_buf):
        for u in range(CK // 16):
            pvec = p_v[ci, pl.ds(u * 16, 16)]
            for j in range(16):
                pj = pvec[j]
                row = u * 16 + j
                for t in range(D // 16):
                    sl = pl.ds(t * 16, 16)
                    rows_buf[row, sl] = rows_buf[row, sl] * pj

    def _gather_start(ci, k):
        pltpu.async_copy(h_hbm.at[src_v.at[ci]], rows[k], gsem[k])

    def _gather_wait(ci, k):
        pltpu.make_async_copy(h_hbm.at[src_v.at[ci]], rows[k],
                              gsem[k]).wait()

    def _scat_start(ci, k):
        pltpu.async_copy(rows[k], acc.at[dst_v.at[ci]], ssem[k], add=True)

    def _scat_wait(ci, k):
        pltpu.make_async_copy(rows[k], acc.at[dst_v.at[ci]], ssem[k]).wait()

    def _group(g, carry):
        pltpu.sync_copy(src_hbm.at[wid, g], src_v)
        pltpu.sync_copy(dst_hbm.at[wid, g], dst_v)
        pltpu.sync_copy(p_hbm.at[wid, g], p_v)
        _gather_start(0, 0)

        def _pair(q, carry2):
            c0 = 2 * q
            _gather_wait(c0, 0)

            @pl.when(q == 0)
            def _first():
                _gather_start(1, 1)

            @pl.when(q > 0)
            def _rf0():
                _scat_wait(c0 + 1, 1)
                _gather_start(c0 + 1, 1)

            _scale(c0, rows[0])
            _scat_start(c0, 0)

            c1 = c0 + 1
            _gather_wait(c1, 1)

            @pl.when(c1 + 1 < SGB)
            def _rf1():
                _scat_wait(c1 + 1, 0)
                _gather_start(c1 + 1, 0)

            _scale(c1, rows[1])
            _scat_start(c1, 1)
            return carry2

        lax.fori_loop(0, SGB // 2, _pair, 0)
        if SGB % 2:
            c = SGB - 1
            _gather_wait(c, 0)
            _scale(c, rows[0])
            _scat_start(c, 0)
        _scat_wait(0, 0)
        _scat_wait(1, 1)
        return carry

    lax.fori_loop(0, NSGB, _group, 0)

    plsc.subcore_barrier()
    pltpu.sync_copy(acc.at[pl.ds(base, STRIPE)],
                    parts_hbm.at[cid, pl.ds(base, STRIPE)])

    @pl.when(sid == NS - 1)
    def _ctail():
        pltpu.sync_copy(acc.at[pl.ds(NS * STRIPE, N - NS * STRIPE)],
                        parts_hbm.at[cid, pl.ds(NS * STRIPE, N - NS * STRIPE)])


def _k2b(src4, dst4, p4, h):
    mesh = plsc.VectorSubcoreMesh(core_axis_name="c", subcore_axis_name="s")
    f = pl.kernel(
        _k2b_body,
        out_type=jax.ShapeDtypeStruct((NC, N, D), jnp.float32),
        mesh=mesh,
        scratch_types=[
            pltpu.VMEM((SGB, CK), jnp.int32),
            pltpu.VMEM((SGB, CK), jnp.int32),
            pltpu.VMEM((SGB, CK), jnp.float32),
            pltpu.VMEM((CK, D), jnp.float32),
            pltpu.VMEM((CK, D), jnp.float32),
            pltpu.VMEM_SHARED((N, D), jnp.float32),
            pltpu.SemaphoreType.DMA,
            pltpu.SemaphoreType.DMA,
            pltpu.SemaphoreType.DMA,
            pltpu.SemaphoreType.DMA,
        ],
        compiler_params=pltpu.CompilerParams(needs_layout_passes=False),
    )
    return f(src4, dst4, p4, h)


def _k3_body(den_ref, parts_ref, out_ref):
    dsum = jnp.sum(den_ref[:, 0, :], axis=0)
    dinv = 1.0 / (dsum + jnp.float32(1e-16))
    out_ref[...] = (parts_ref[0] + parts_ref[1]) * dinv[:, None]


def _k3(den, parts):
    return pl.pallas_call(
        _k3_body,
        out_shape=jax.ShapeDtypeStruct((N, D), jnp.float32),
    )(den, parts)


@jax.jit
def kernel(x, edge_index, W, a_src, a_dst):
    pad = jnp.zeros((NW, EPWP - EPW), jnp.int32)
    srcp = jnp.concatenate([edge_index[0].reshape(NW, EPW), pad], axis=1)
    dstp = jnp.concatenate([edge_index[1].reshape(NW, EPW), pad], axis=1)
    src_a = srcp.reshape(NW, NSG, GE // 128, 128)
    dst_a = dstp.reshape(NW, NSG, GE // 128, 128)
    src_b = edge_index[0].reshape(NW, NSGB, SGB, CK)
    dst_b = edge_index[1].reshape(NW, NSGB, SGB, CK)

    h, s, d, m = _k1(x, W, a_src, a_dst)
    p4, den = _k2a(src_a, dst_a, s.reshape(N), d.reshape(N), m)
    p_b = p4.reshape(NW, EPWP)[:, :EPW].reshape(NW, NSGB, SGB, CK)
    parts = _k2b(src_b, dst_b, p_b, h)
    return _k3(den, parts)


# speedup vs baseline: 1.0113x; 1.0113x over previous
import jax
import jax.numpy as jnp
from jax import lax
from jax.experimental import pallas as pl
from jax.experimental.pallas import tpu as pltpu
from jax.experimental.pallas import tpu_sc as plsc

N = 10000
E = 320000
D = 128

NC = 2
NS = 16
NW = NC * NS
EPW = E // NW
NSG = 10
GE = 1024
EPWP = NSG * GE
NSGB = 5
GEB = 2000
CK = 80
SGB = GEB // CK
STRIPE = 624
ZR = 16


def _k1_body(x_ref, w_ref, asrc_ref, adst_ref, h_ref, s_ref, d_ref, m_ref):
    i = pl.program_id(0)
    h = jnp.dot(x_ref[...], w_ref[...], preferred_element_type=jnp.float32)
    h_ref[...] = h
    s = jnp.dot(h, asrc_ref[...], preferred_element_type=jnp.float32)
    d = jnp.dot(h, adst_ref[...], preferred_element_type=jnp.float32)
    s_ref[...] = s[:, None]
    d_ref[...] = d[:, None]
    bs = jnp.max(s)
    bd = jnp.max(d)

    @pl.when(i == 0)
    def _init():
        m_ref[0, 0] = bs
        m_ref[0, 1] = bd

    @pl.when(i > 0)
    def _acc():
        m_ref[0, 0] = jnp.maximum(m_ref[0, 0], bs)
        m_ref[0, 1] = jnp.maximum(m_ref[0, 1], bd)


_K1_BLK = 1000


def _k1(x, W, a_src, a_dst):
    return pl.pallas_call(
        _k1_body,
        grid=(N // _K1_BLK,),
        in_specs=[
            pl.BlockSpec((_K1_BLK, D), lambda i: (i, 0)),
            pl.BlockSpec((D, D), lambda i: (0, 0)),
            pl.BlockSpec((D,), lambda i: (0,)),
            pl.BlockSpec((D,), lambda i: (0,)),
        ],
        out_specs=[
            pl.BlockSpec((_K1_BLK, D), lambda i: (i, 0)),
            pl.BlockSpec((_K1_BLK, 1), lambda i: (i, 0)),
            pl.BlockSpec((_K1_BLK, 1), lambda i: (i, 0)),
            pl.BlockSpec((1, 16), lambda i: (0, 0), memory_space=pltpu.SMEM),
        ],
        out_shape=[
            jax.ShapeDtypeStruct((N, D), jnp.float32),
            jax.ShapeDtypeStruct((N, 1), jnp.float32),
            jax.ShapeDtypeStruct((N, 1), jnp.float32),
            jax.ShapeDtypeStruct((1, 16), jnp.float32),
        ],
    )(x, W, a_src, a_dst)


def _k2a_body(src_hbm, dst_hbm, s_hbm, d_hbm, m_hbm,
              p_hbm, den_hbm,
              src_v, dst_v, s_v, d_v, den_v, p_v, m_v):
    cid = lax.axis_index("c")
    sid = lax.axis_index("s")
    wid = sid * NC + cid

    pltpu.sync_copy(s_hbm, s_v)
    pltpu.sync_copy(d_hbm, d_v)
    pltpu.sync_copy(m_hbm, m_v)
    mvec = m_v[0, pl.ds(0, 16)]
    c_sh = jnp.maximum(mvec[0] + mvec[1], 0.0)

    zeros16 = jnp.zeros((16,), jnp.float32)

    def _zden(i, carry):
        den_v[0, pl.ds(i * 16, 16)] = zeros16
        return carry
    lax.fori_loop(0, N // 16, _zden, 0)

    lane = lax.iota(jnp.int32, 16)

    def _group(g, carry):
        pltpu.sync_copy(src_hbm.at[wid, g], src_v)
        pltpu.sync_copy(dst_hbm.at[wid, g], dst_v)

        def _vec(q, carry2):
            r = q // 8
            c0 = (q % 8) * 16
            sl = pl.ds(c0, 16)
            si = src_v[r, sl]
            di = dst_v[r, sl]
            z = plsc.load_gather(s_v, [si]) + plsc.load_gather(d_v, [di])
            e = jnp.where(z > 0, z, z * jnp.float32(0.2))
            eidx = g * GE + q * 16 + lane
            p = jnp.where(eidx < EPW, jnp.exp(e - c_sh), 0.0)
            p_v[r, sl] = p
            plsc.addupdate_scatter(den_v.at[0], [di], p)
            return carry2
        lax.fori_loop(0, GE // 16, _vec, 0)

        pltpu.sync_copy(p_v, p_hbm.at[wid, g])
        return carry

    lax.fori_loop(0, NSG, _group, 0)
    pltpu.sync_copy(den_v, den_hbm.at[wid])


def _k2a(src4, dst4, s, d, m):
    mesh = plsc.VectorSubcoreMesh(core_axis_name="c", subcore_axis_name="s")
    f = pl.kernel(
        _k2a_body,
        out_type=[
            jax.ShapeDtypeStruct((NW, NSG, GE // 128, 128), jnp.float32),
            jax.ShapeDtypeStruct((NW, 1, N), jnp.float32),
        ],
        mesh=mesh,
        scratch_types=[
            pltpu.VMEM((GE // 128, 128), jnp.int32),
            pltpu.VMEM((GE // 128, 128), jnp.int32),
            pltpu.VMEM((N,), jnp.float32),
            pltpu.VMEM((N,), jnp.float32),
            pltpu.VMEM((1, N), jnp.float32),
            pltpu.VMEM((GE // 128, 128), jnp.float32),
            pltpu.VMEM((1, 16), jnp.float32),
        ],
        compiler_params=pltpu.CompilerParams(needs_layout_passes=False),
    )
    return f(src4, dst4, s, d, m)


def _k2b_body(src_hbm, dst_hbm, p_hbm, h_hbm,
              parts_hbm,
              src_v, dst_v, p_v,
              rows0, rows1, acc,
              g0, g1, s0, s1):
    cid = lax.axis_index("c")
    sid = lax.axis_index("s")
    wid = sid * NC + cid
    base = pl.multiple_of(sid * STRIPE, 8)
    rows = [rows0, rows1]
    gsem = [g0, g1]
    ssem = [s0, s1]

    zeros16 = jnp.zeros((16,), jnp.float32)

    def _zbuf(i, carry):
        for t in range(D // 16):
            rows0[i, pl.ds(t * 16, 16)] = zeros16
        return carry
    lax.fori_loop(0, ZR, _zbuf, 0)
    zsrc = rows0.at[pl.ds(0, ZR)]

    def _zacc(r, carry):
        off = pl.multiple_of(base + r * ZR, 8)
        pltpu.sync_copy(zsrc, acc.at[pl.ds(off, ZR)])
        return carry
    lax.fori_loop(0, STRIPE // ZR, _zacc, 0)

    @pl.when(sid == NS - 1)
    def _ztail():
        pltpu.sync_copy(zsrc, acc.at[pl.ds(NS * STRIPE, N - NS * STRIPE)])

    plsc.subcore_barrier()

    def _scale(ci, rows_buf):
        for u in range(CK // 16):
            pvec = p_v[ci, pl.ds(u * 16, 16)]
            for j in range(16):
                pj = pvec[j]
                row = u * 16 + j
                for t in range(D // 16):
                    sl = pl.ds(t * 16, 16)
                    rows_buf[row, sl] = rows_buf[row, sl] * pj

    def _gather_start(ci, k):
        pltpu.async_copy(h_hbm.at[src_v.at[ci]], rows[k], gsem[k])

    def _gather_wait(ci, k):
        pltpu.make_async_copy(h_hbm.at[src_v.at[ci]], rows[k],
                              gsem[k]).wait()

    def _scat_start(ci, k):
        pltpu.async_copy(rows[k], acc.at[dst_v.at[ci]], ssem[k], add=True)

    def _scat_wait(ci, k):
        pltpu.make_async_copy(rows[k], acc.at[dst_v.at[ci]], ssem[k]).wait()

    def _group(g, carry):
        pltpu.sync_copy(src_hbm.at[wid, g], src_v)
        pltpu.sync_copy(dst_hbm.at[wid, g], dst_v)
        pltpu.sync_copy(p_hbm.at[wid, g], p_v)
        _gather_start(0, 0)

        def _pair(q, carry2):
            c0 = 2 * q
            _gather_wait(c0, 0)

            @pl.when(q == 0)
            def _first():
                _gather_start(1, 1)

            @pl.when(q > 0)
            def _rf0():
                _scat_wait(c0 + 1, 1)
                _gather_start(c0 + 1, 1)

            _scale(c0, rows[0])
            _scat_start(c0, 0)

            c1 = c0 + 1
            _gather_wait(c1, 1)

            @pl.when(c1 + 1 < SGB)
            def _rf1():
                _scat_wait(c1 + 1, 0)
                _gather_start(c1 + 1, 0)

            _scale(c1, rows[1])
            _scat_start(c1, 1)
            return carry2

        lax.fori_loop(0, SGB // 2, _pair, 0)
        if SGB % 2:
            c = SGB - 1
            _gather_wait(c, 0)
            _scale(c, rows[0])
            _scat_start(c, 0)
        _scat_wait(0, 0)
        _scat_wait(1, 1)
        return carry

    lax.fori_loop(0, NSGB, _group, 0)

    plsc.subcore_barrier()
    pltpu.sync_copy(acc.at[pl.ds(base, STRIPE)],
                    parts_hbm.at[cid, pl.ds(base, STRIPE)])

    @pl.when(sid == NS - 1)
    def _ctail():
        pltpu.sync_copy(acc.at[pl.ds(NS * STRIPE, N - NS * STRIPE)],
                        parts_hbm.at[cid, pl.ds(NS * STRIPE, N - NS * STRIPE)])


def _k2b(src4, dst4, p4, h):
    mesh = plsc.VectorSubcoreMesh(core_axis_name="c", subcore_axis_name="s")
    f = pl.kernel(
        _k2b_body,
        out_type=jax.ShapeDtypeStruct((NC, N, D), jnp.float32),
        mesh=mesh,
        scratch_types=[
            pltpu.VMEM((SGB, CK), jnp.int32),
            pltpu.VMEM((SGB, CK), jnp.int32),
            pltpu.VMEM((SGB, CK), jnp.float32),
            pltpu.VMEM((CK, D), jnp.float32),
            pltpu.VMEM((CK, D), jnp.float32),
            pltpu.VMEM_SHARED((N, D), jnp.float32),
            pltpu.SemaphoreType.DMA,
            pltpu.SemaphoreType.DMA,
            pltpu.SemaphoreType.DMA,
            pltpu.SemaphoreType.DMA,
        ],
        compiler_params=pltpu.CompilerParams(needs_layout_passes=False),
    )
    return f(src4, dst4, p4, h)


def _k3_body(den_ref, parts_ref, out_ref):
    dsum = jnp.sum(den_ref[:, 0, :], axis=0)
    dinv = 1.0 / (dsum + jnp.float32(1e-16))
    out_ref[...] = (parts_ref[0] + parts_ref[1]) * dinv[:, None]


def _k3(den, parts):
    return pl.pallas_call(
        _k3_body,
        out_shape=jax.ShapeDtypeStruct((N, D), jnp.float32),
    )(den, parts)


@jax.jit
def kernel(x, edge_index, W, a_src, a_dst):
    pad = jnp.zeros((NW, EPWP - EPW), jnp.int32)
    srcp = jnp.concatenate([edge_index[0].reshape(NW, EPW), pad], axis=1)
    dstp = jnp.concatenate([edge_index[1].reshape(NW, EPW), pad], axis=1)
    src_a = srcp.reshape(NW, NSG, GE // 128, 128)
    dst_a = dstp.reshape(NW, NSG, GE // 128, 128)
    src_b = edge_index[0].reshape(NW, NSGB, SGB, CK)
    dst_b = edge_index[1].reshape(NW, NSGB, SGB, CK)

    h, s, d, m = _k1(x, W, a_src, a_dst)
    p4, den = _k2a(src_a, dst_a, s.reshape(N), d.reshape(N), m)
    p_b = p4.reshape(NW, EPWP)[:, :EPW].reshape(NW, NSGB, SGB, CK)
    parts = _k2b(src_b, dst_b, p_b, h)
    return _k3(den, parts)
